# baseline (speedup 1.0000x reference)
import functools

import jax
import jax.numpy as jnp
from jax import lax
from jax.experimental import pallas as pl
from jax.experimental.pallas import tpu as pltpu
from jax.experimental.pallas import tpu_sc as plsc

D_MODEL = 1024
D_FF = 2048
NUM_EXPERTS = 8
TOP_K = 2
TOKENS = 4096
HALF = TOKENS // 2
ROW_TILE = 256
NUM_TILES = 24
RSORT = NUM_TILES * ROW_TILE
NW = 32
TPW = HALF // NW


def _route_body(x_ref, wg_ref, bg_ref, pos_ref, sb_ref, eot_ref):
    xb = x_ref[...].astype(jnp.bfloat16)
    scores = jnp.dot(xb, wg_ref[...].astype(jnp.bfloat16),
                     preferred_element_type=jnp.float32) + bg_ref[...]
    cols = lax.broadcasted_iota(jnp.int32, (TOKENS, NUM_EXPERTS), 1)
    m = jnp.max(scores, axis=1, keepdims=True)
    p = jnp.exp(scores - m)
    p = p / jnp.sum(p, axis=1, keepdims=True)
    v0 = jnp.max(p, axis=1, keepdims=True)
    a0 = jnp.min(jnp.where(p >= v0, cols, NUM_EXPERTS), axis=1, keepdims=True)
    p1m = jnp.where(cols == a0, -jnp.inf, p)
    v1 = jnp.max(p1m, axis=1, keepdims=True)
    a1 = jnp.min(jnp.where(p1m >= v1, cols, NUM_EXPERTS), axis=1, keepdims=True)
    oh0 = (cols == a0).astype(jnp.float32)
    oh1 = (cols == a1).astype(jnp.float32)
    sel = oh0 + oh1
    rows = lax.broadcasted_iota(jnp.int32, (TOKENS, 1), 0)
    rows_h = rows - HALF * (rows >= HALF).astype(jnp.int32)
    c = sel
    d = 1
    while d < HALF:
        z = jnp.zeros((d, NUM_EXPERTS), jnp.float32)
        shifted = jnp.concatenate([z, c[:TOKENS - d, :]], axis=0)
        c = c + jnp.where(rows_h >= d, shifted, 0.0)
        d *= 2
    rank = c - sel
    counts2 = jnp.concatenate([c[HALF - 1:HALF, :], c[TOKENS - 1:TOKENS, :]],
                              axis=0)
    cp2 = jnp.floor((counts2 + (ROW_TILE - 1)) / ROW_TILE) * ROW_TILE
    rr = lax.broadcasted_iota(jnp.int32, (NUM_EXPERTS, NUM_EXPERTS), 0)
    cc = lax.broadcasted_iota(jnp.int32, (NUM_EXPERTS, NUM_EXPERTS), 1)
    tri = (rr < cc).astype(jnp.float32)
    op2 = jnp.dot(cp2, tri, preferred_element_type=jnp.float32)
    opend2 = op2 + cp2
    ident = (rr == cc).astype(jnp.float32)
    opend_t = lax.dot_general(ident, opend2, (((1,), (1,)), ((), ())),
                              preferred_element_type=jnp.float32)
    lanes48 = lax.broadcasted_iota(jnp.int32, (1, 48), 1)
    in_h1 = lanes48 >= NUM_TILES
    tile_idx = lanes48 - NUM_TILES * in_h1.astype(jnp.int32)
    tile_start = tile_idx.astype(jnp.float32) * ROW_TILE
    opend_sel = jnp.where(in_h1, opend_t[:, 1:2], opend_t[:, 0:1])
    eot48 = jnp.sum((tile_start >= opend_sel).astype(jnp.float32),
                    axis=0, keepdims=True)
    eot48 = jnp.minimum(eot48, NUM_EXPERTS - 1.0)
    eot64 = jnp.concatenate([eot48, jnp.zeros((1, 16), jnp.float32)], axis=1)
    nu0 = jnp.sum(cp2[0:1, :]) * (1.0 / ROW_TILE)
    nu1 = jnp.sum(cp2[1:2, :]) * (1.0 / ROW_TILE)
    lanes64 = lax.broadcasted_iota(jnp.int32, (1, 64), 1)
    eot64 = jnp.where(lanes64 == 62, nu0, eot64)
    eot64 = jnp.where(lanes64 == 63, nu1, eot64)
    eot_ref[...] = eot64.astype(jnp.int32)
    op_bc = jnp.where(rows >= HALF, op2[1:2, :], op2[0:1, :])
    posf = op_bc + rank
    pw0 = jnp.sum(posf * oh0, axis=1, keepdims=True)
    pw1 = jnp.sum(posf * oh1, axis=1, keepdims=True)
    pos_ref[...] = jnp.concatenate([pw0, pw1], axis=1).astype(jnp.int32)
    s0 = jnp.sum(p * oh0, axis=1, keepdims=True)
    s1 = jnp.sum(p * oh1, axis=1, keepdims=True)
    sb_ref[...] = jnp.concatenate(
        [jnp.broadcast_to(s0, (TOKENS, 16)),
         jnp.broadcast_to(s1, (TOKENS, 16))], axis=0)


def _route(x2, Wg, bg):
    return pl.pallas_call(
        _route_body,
        out_shape=[
            jax.ShapeDtypeStruct((TOKENS, 2), jnp.int32),
            jax.ShapeDtypeStruct((2 * TOKENS, 16), jnp.float32),
            jax.ShapeDtypeStruct((1, 64), jnp.int32),
        ],
    )(x2, Wg, bg.reshape(1, NUM_EXPERTS))


_DCH = 32
_DNC = TPW // _DCH


def _dispatch_body(h, x_hbm, pos_hbm, xs_hbm, xb0, xb1, *rest):
    idx = rest[:2 * _DNC]
    isem, lsem0, lsem1, ssem0, ssem1 = rest[2 * _DNC:]
    xbufs = (xb0, xb1)
    lsems = (lsem0, lsem1)
    ssems = (ssem0, ssem1)
    wid = lax.axis_index("s") * 2 + lax.axis_index("c")
    base0 = h * HALF + wid * TPW
    icp = []
    for ci in range(_DNC):
        for k in range(2):
            icp.append(pltpu.async_copy(
                pos_hbm.at[k, pl.ds(base0 + ci * _DCH, _DCH)],
                idx[k * _DNC + ci], isem))
    loads = [None, None]
    scats = [None, None, None, None]
    for ci in range(2):
        loads[ci] = pltpu.async_copy(
            x_hbm.at[pl.ds(base0 + ci * _DCH, _DCH)], xbufs[ci], lsems[ci])
    for cpy in icp:
        cpy.wait()
    for ci in range(_DNC):
        b = ci % 2
        loads[b].wait()
        scats[2 * b] = pltpu.async_copy(
            xbufs[b], xs_hbm.at[idx[ci]], ssems[b])
        scats[2 * b + 1] = pltpu.async_copy(
            xbufs[b], xs_hbm.at[idx[_DNC + ci]], ssems[b])
        if ci + 2 < _DNC:
            scats[2 * b].wait()
            scats[2 * b + 1].wait()
            loads[b] = pltpu.async_copy(
                x_hbm.at[pl.ds(base0 + (ci + 2) * _DCH, _DCH)],
                xbufs[b], lsems[b])
    for b in range(min(2, _DNC)):
        scats[2 * b].wait()
        scats[2 * b + 1].wait()


def _dispatch(x2, pos_t, h):
    mesh = plsc.VectorSubcoreMesh(core_axis_name="c", subcore_axis_name="s")
    return pl.kernel(
        functools.partial(_dispatch_body, h),
        out_type=jax.ShapeDtypeStruct((RSORT, D_MODEL), jnp.float32),
        mesh=mesh,
        scratch_types=(
            [pltpu.VMEM((_DCH, D_MODEL), jnp.float32)] * 2
            + [pltpu.VMEM((_DCH,), jnp.int32)] * (2 * _DNC)
            + [pltpu.SemaphoreType.DMA] * 5
        ),
    )(x2, pos_t)


def _ffn_body(eot_ref, xs_ref, w1_ref, b1_ref, w2_ref, b2_ref, out_ref):
    i = pl.program_id(0)

    @pl.when(i < eot_ref[NUM_TILES])
    def _():
        h = jnp.dot(xs_ref[...], w1_ref[0], preferred_element_type=jnp.float32)
        h = jnp.maximum(h + b1_ref[0], 0.0)
        out_ref[...] = jnp.dot(h, w2_ref[0],
                               preferred_element_type=jnp.float32) + b2_ref[0]


def _ffn(eot, xs, W1, b1r, W2, b2r):
    grid_spec = pltpu.PrefetchScalarGridSpec(
        num_scalar_prefetch=1,
        grid=(NUM_TILES,),
        in_specs=[
            pl.BlockSpec((ROW_TILE, D_MODEL), lambda i, eot: (i, 0)),
            pl.BlockSpec((1, D_MODEL, D_FF), lambda i, eot: (eot[i], 0, 0)),
            pl.BlockSpec((1, 1, D_FF), lambda i, eot: (eot[i], 0, 0)),
            pl.BlockSpec((1, D_FF, D_MODEL), lambda i, eot: (eot[i], 0, 0)),
            pl.BlockSpec((1, 1, D_MODEL), lambda i, eot: (eot[i], 0, 0)),
        ],
        out_specs=pl.BlockSpec((ROW_TILE, D_MODEL), lambda i, eot: (i, 0)),
    )
    return pl.pallas_call(
        _ffn_body,
        grid_spec=grid_spec,
        out_shape=jax.ShapeDtypeStruct((RSORT, D_MODEL), jnp.float32),
        compiler_params=pltpu.CompilerParams(
            dimension_semantics=("arbitrary",),
        ),
    )(eot, xs, W1, b1r, W2, b2r)


_CCH = 16
_CNC = TPW // _CCH


def _combine_body(h, ys_hbm, pos_hbm, sb_hbm, out_hbm,
                  y0a, y1a, y0b, y1b,
                  sbuf0, sbuf1, idx0, idx1,
                  isem, gsa, gsb, osa, osb):
    wid = lax.axis_index("s") * 2 + lax.axis_index("c")
    gbase = h * HALF + wid * TPW
    obase = wid * TPW
    i0 = pltpu.async_copy(pos_hbm.at[0, pl.ds(gbase, TPW)], idx0, isem)
    i1 = pltpu.async_copy(pos_hbm.at[1, pl.ds(gbase, TPW)], idx1, isem)
    i2 = pltpu.async_copy(sb_hbm.at[pl.ds(gbase, TPW)], sbuf0, isem)
    i3 = pltpu.async_copy(sb_hbm.at[pl.ds(TOKENS + gbase, TPW)], sbuf1, isem)
    i0.wait(); i1.wait(); i2.wait(); i3.wait()
    ybufs = ((y0a, y1a), (y0b, y1b))
    gsems = (gsa, gsb)
    osems = (osa, osb)
    gaths = [None, None]
    outs = [None, None]

    def gather(ci, b):
        sl = pl.ds(ci * _CCH, _CCH)
        g0 = pltpu.async_copy(ys_hbm.at[idx0.at[sl]], ybufs[b][0], gsems[b])
        g1 = pltpu.async_copy(ys_hbm.at[idx1.at[sl]], ybufs[b][1], gsems[b])
        return (g0, g1)

    gaths[0] = gather(0, 0)
    gaths[1] = gather(1, 1)
    for ci in range(_CNC):
        b = ci % 2
        gaths[b][0].wait()
        gaths[b][1].wait()
        y0, y1 = ybufs[b]

        def row_body(j, carry, y0=y0, y1=y1, ci=ci):
            sv0 = sbuf0[pl.ds(ci * _CCH + j, 1), pl.ds(0, 16)]
            sv1 = sbuf1[pl.ds(ci * _CCH + j, 1), pl.ds(0, 16)]
            s0v = sv0.reshape((16,))
            s1v = sv1.reshape((16,))
            for col in range(D_MODEL // 16):
                sl2 = pl.ds(col * 16, 16)
                y0[j, sl2] = y0[j, sl2] * s0v + y1[j, sl2] * s1v
            return carry

        lax.fori_loop(0, _CCH, row_body, 0)
        outs[b] = pltpu.async_copy(
            y0, out_hbm.at[pl.ds(obase + ci * _CCH, _CCH)], osems[b])
        if ci + 2 < _CNC:
            outs[b].wait()
            gaths[b] = gather(ci + 2, b)
    for b in range(min(2, _CNC)):
        outs[b].wait()


def _combine(ys, pos_t, sb, h):
    mesh = plsc.VectorSubcoreMesh(core_axis_name="c", subcore_axis_name="s")
    return pl.kernel(
        functools.partial(_combine_body, h),
        out_type=jax.ShapeDtypeStruct((HALF, D_MODEL), jnp.float32),
        mesh=mesh,
        scratch_types=(
            [pltpu.VMEM((_CCH, D_MODEL), jnp.float32)] * 4
            + [pltpu.VMEM((TPW, 16), jnp.float32)] * 2
            + [pltpu.VMEM((TPW,), jnp.int32)] * 2
            + [pltpu.SemaphoreType.DMA] * 5
        ),
    )(ys, pos_t, sb)


@jax.jit
def kernel(x, W1, b1, W2, b2, Wg, bg):
    B, N, D = x.shape
    x2 = x.reshape(B * N, D)
    pos01, sb, eot64 = _route(x2, Wg, bg)
    pos_t = pos01.T
    e64 = eot64.reshape(64)
    eh0 = jnp.concatenate([e64[0:NUM_TILES], e64[62:63]])
    eh1 = jnp.concatenate([e64[NUM_TILES:2 * NUM_TILES], e64[63:64]])
    b1r = b1.reshape(NUM_EXPERTS, 1, D_FF)
    b2r = b2.reshape(NUM_EXPERTS, 1, D_MODEL)
    xs0 = _dispatch(x2, pos_t, 0)
    xs1 = _dispatch(x2, pos_t, 1)
    ys0 = _ffn(eh0, xs0, W1, b1r, W2, b2r)
    ys1 = _ffn(eh1, xs1, W1, b1r, W2, b2r)
    o0 = _combine(ys0, pos_t, sb, 0)
    o1 = _combine(ys1, pos_t, sb, 1)
    return jnp.concatenate([o0, o1], axis=0).reshape(B, N, D)


# speedup vs baseline: 1.2434x; 1.2434x over previous
import functools

import jax
import jax.numpy as jnp
from jax import lax
from jax.experimental import pallas as pl
from jax.experimental.pallas import tpu as pltpu
from jax.experimental.pallas import tpu_sc as plsc

D_MODEL = 1024
D_FF = 2048
NUM_EXPERTS = 8
TOP_K = 2
TOKENS = 4096
ROW_TILE = 256
NUM_TILES = 40
RSORT = NUM_TILES * ROW_TILE
NW = 32
TPW = TOKENS // NW


def _route_body(x_ref, wg_ref, bg_ref, pos_ref, sb_ref, eot_ref):
    xb = x_ref[...].astype(jnp.bfloat16)
    scores = jnp.dot(xb, wg_ref[...].astype(jnp.bfloat16),
                     preferred_element_type=jnp.float32) + bg_ref[...]
    cols = lax.broadcasted_iota(jnp.int32, (TOKENS, NUM_EXPERTS), 1)
    m = jnp.max(scores, axis=1, keepdims=True)
    p = jnp.exp(scores - m)
    p = p / jnp.sum(p, axis=1, keepdims=True)
    v0 = jnp.max(p, axis=1, keepdims=True)
    a0 = jnp.min(jnp.where(p >= v0, cols, NUM_EXPERTS), axis=1, keepdims=True)
    p1m = jnp.where(cols == a0, -jnp.inf, p)
    v1 = jnp.max(p1m, axis=1, keepdims=True)
    a1 = jnp.min(jnp.where(p1m >= v1, cols, NUM_EXPERTS), axis=1, keepdims=True)
    oh0 = (cols == a0).astype(jnp.float32)
    oh1 = (cols == a1).astype(jnp.float32)
    sel = oh0 + oh1
    c = sel
    d = 1
    while d < TOKENS:
        z = jnp.zeros((d, NUM_EXPERTS), jnp.float32)
        c = c + jnp.concatenate([z, c[:TOKENS - d, :]], axis=0)
        d *= 2
    rank = c - sel
    counts = jnp.sum(sel, axis=0, keepdims=True)
    cp = jnp.floor((counts + (ROW_TILE - 1)) / ROW_TILE) * ROW_TILE
    rr = lax.broadcasted_iota(jnp.int32, (NUM_EXPERTS, NUM_EXPERTS), 0)
    cc = lax.broadcasted_iota(jnp.int32, (NUM_EXPERTS, NUM_EXPERTS), 1)
    tri = (rr < cc).astype(jnp.float32)
    op = jnp.dot(cp, tri, preferred_element_type=jnp.float32)
    opend = op + cp
    ident = (rr == cc).astype(jnp.float32)
    opend_t = lax.dot_general(ident, opend, (((1,), (1,)), ((), ())),
                              preferred_element_type=jnp.float32)
    tile_start = lax.broadcasted_iota(
        jnp.int32, (NUM_EXPERTS, 64), 1).astype(jnp.float32) * ROW_TILE
    eot = jnp.sum((tile_start >= opend_t).astype(jnp.float32),
                  axis=0, keepdims=True)
    eot = jnp.minimum(eot, NUM_EXPERTS - 1.0)
    lanes64 = lax.broadcasted_iota(jnp.int32, (1, 64), 1)
    n_used = jnp.sum(cp) / ROW_TILE
    eot_ref[...] = jnp.where(lanes64 == 63, n_used, eot).astype(jnp.int32)
    posf = op + rank
    pw0 = jnp.sum(posf * oh0, axis=1, keepdims=True)
    pw1 = jnp.sum(posf * oh1, axis=1, keepdims=True)
    pos_ref[...] = jnp.concatenate([pw0, pw1], axis=1).astype(jnp.int32)
    s0 = jnp.sum(p * oh0, axis=1, keepdims=True)
    s1 = jnp.sum(p * oh1, axis=1, keepdims=True)
    sb_ref[...] = jnp.concatenate(
        [jnp.broadcast_to(s0, (TOKENS, 16)),
         jnp.broadcast_to(s1, (TOKENS, 16))], axis=0)


def _route(x2, Wg, bg):
    return pl.pallas_call(
        _route_body,
        out_shape=[
            jax.ShapeDtypeStruct((TOKENS, 2), jnp.int32),
            jax.ShapeDtypeStruct((2 * TOKENS, 16), jnp.float32),
            jax.ShapeDtypeStruct((1, 64), jnp.int32),
        ],
    )(x2, Wg, bg.reshape(1, NUM_EXPERTS))


_DCH = 32
_DNC = TPW // _DCH


def _dispatch_body(x_hbm, pos_hbm, xs_hbm, xb0, xb1, *rest):
    idx = rest[:2 * _DNC]
    isem, lsem0, lsem1, ssem0, ssem1 = rest[2 * _DNC:]
    xbufs = (xb0, xb1)
    lsems = (lsem0, lsem1)
    ssems = (ssem0, ssem1)
    wid = lax.axis_index("s") * 2 + lax.axis_index("c")
    base0 = wid * TPW
    icp = []
    for ci in range(_DNC):
        for k in range(2):
            icp.append(pltpu.async_copy(
                pos_hbm.at[k, pl.ds(base0 + ci * _DCH, _DCH)],
                idx[k * _DNC + ci], isem))
    loads = [None, None]
    scats = [None] * _DNC
    for ci in range(2):
        loads[ci] = pltpu.async_copy(
            x_hbm.at[pl.ds(base0 + ci * _DCH, _DCH)], xbufs[ci], lsems[ci])
    for cpy in icp:
        cpy.wait()
    for ci in range(_DNC):
        b = ci % 2
        loads[b].wait()
        scats[ci] = (
            pltpu.async_copy(xbufs[b], xs_hbm.at[idx[ci]], ssems[b]),
            pltpu.async_copy(xbufs[b], xs_hbm.at[idx[_DNC + ci]], ssems[b]),
        )
        if ci - 1 >= 0 and ci + 1 < _DNC:
            scats[ci - 1][0].wait()
            scats[ci - 1][1].wait()
            loads[(ci + 1) % 2] = pltpu.async_copy(
                x_hbm.at[pl.ds(base0 + (ci + 1) * _DCH, _DCH)],
                xbufs[(ci + 1) % 2], lsems[(ci + 1) % 2])
    for ci in (_DNC - 2, _DNC - 1):
        scats[ci][0].wait()
        scats[ci][1].wait()


def _dispatch(x2, pos_t):
    mesh = plsc.VectorSubcoreMesh(core_axis_name="c", subcore_axis_name="s")
    return pl.kernel(
        _dispatch_body,
        out_type=jax.ShapeDtypeStruct((RSORT, D_MODEL), jnp.float32),
        mesh=mesh,
        scratch_types=(
            [pltpu.VMEM((_DCH, D_MODEL), jnp.float32)] * 2
            + [pltpu.VMEM((_DCH,), jnp.int32)] * (2 * _DNC)
            + [pltpu.SemaphoreType.DMA] * 5
        ),
    )(x2, pos_t)


def _ffn_body(eot_ref, xs_ref, w1_ref, b1_ref, w2_ref, b2_ref, out_ref):
    i = pl.program_id(0)

    @pl.when(i < eot_ref[63])
    def _():
        h = jnp.dot(xs_ref[...], w1_ref[0], preferred_element_type=jnp.float32)
        h = jnp.maximum(h + b1_ref[0], 0.0)
        out_ref[...] = jnp.dot(h, w2_ref[0],
                               preferred_element_type=jnp.float32) + b2_ref[0]


def _ffn(eot, xs, W1b, b1r, W2b, b2r):
    grid_spec = pltpu.PrefetchScalarGridSpec(
        num_scalar_prefetch=1,
        grid=(NUM_TILES,),
        in_specs=[
            pl.BlockSpec((ROW_TILE, D_MODEL), lambda i, eot: (i, 0)),
            pl.BlockSpec((1, D_MODEL, D_FF), lambda i, eot: (eot[i], 0, 0)),
            pl.BlockSpec((1, 1, D_FF), lambda i, eot: (eot[i], 0, 0)),
            pl.BlockSpec((1, D_FF, D_MODEL), lambda i, eot: (eot[i], 0, 0)),
            pl.BlockSpec((1, 1, D_MODEL), lambda i, eot: (eot[i], 0, 0)),
        ],
        out_specs=pl.BlockSpec((ROW_TILE, D_MODEL), lambda i, eot: (i, 0)),
    )
    return pl.pallas_call(
        _ffn_body,
        grid_spec=grid_spec,
        out_shape=jax.ShapeDtypeStruct((RSORT, D_MODEL), jnp.float32),
        compiler_params=pltpu.CompilerParams(
            dimension_semantics=("arbitrary",),
        ),
    )(eot, xs, W1b, b1r, W2b, b2r)


_CCH = 16


_CNC = TPW // _CCH


def _combine_body(ys_hbm, pos_hbm, sb_hbm, out_hbm,
                  y0a, y1a, y0b, y1b,
                  sbuf0, sbuf1, idx0, idx1,
                  isem, gsa, gsb, osa, osb):
    wid = lax.axis_index("s") * 2 + lax.axis_index("c")
    base0 = wid * TPW
    i0 = pltpu.async_copy(pos_hbm.at[0, pl.ds(base0, TPW)], idx0, isem)
    i1 = pltpu.async_copy(pos_hbm.at[1, pl.ds(base0, TPW)], idx1, isem)
    i2 = pltpu.async_copy(sb_hbm.at[pl.ds(base0, TPW)], sbuf0, isem)
    i3 = pltpu.async_copy(sb_hbm.at[pl.ds(TOKENS + base0, TPW)], sbuf1, isem)
    i0.wait(); i1.wait(); i2.wait(); i3.wait()
    ybufs = ((y0a, y1a), (y0b, y1b))
    gsems = (gsa, gsb)
    osems = (osa, osb)
    gaths = [None, None]
    outs = [None, None]

    def gather(ci, b):
        sl = pl.ds(ci * _CCH, _CCH)
        g0 = pltpu.async_copy(ys_hbm.at[idx0.at[sl]], ybufs[b][0], gsems[b])
        g1 = pltpu.async_copy(ys_hbm.at[idx1.at[sl]], ybufs[b][1], gsems[b])
        return (g0, g1)

    gaths[0] = gather(0, 0)
    gaths[1] = gather(1, 1)
    for ci in range(_CNC):
        b = ci % 2
        gaths[b][0].wait()
        gaths[b][1].wait()
        y0, y1 = ybufs[b]

        def row_body(j, carry, y0=y0, y1=y1, ci=ci):
            sv0 = sbuf0[pl.ds(ci * _CCH + j, 1), pl.ds(0, 16)]
            sv1 = sbuf1[pl.ds(ci * _CCH + j, 1), pl.ds(0, 16)]
            s0v = sv0.reshape((16,))
            s1v = sv1.reshape((16,))
            for cc in range(D_MODEL // 16):
                sl2 = pl.ds(cc * 16, 16)
                y0[j, sl2] = y0[j, sl2] * s0v + y1[j, sl2] * s1v
            return carry

        lax.fori_loop(0, _CCH, row_body, 0)
        outs[b] = pltpu.async_copy(
            y0, out_hbm.at[pl.ds(base0 + ci * _CCH, _CCH)], osems[b])
        if ci + 2 < _CNC:
            outs[b].wait()
            gaths[b] = gather(ci + 2, b)
    for b in range(2):
        outs[b].wait()


def _combine(ys, pos_t, sb):
    mesh = plsc.VectorSubcoreMesh(core_axis_name="c", subcore_axis_name="s")
    return pl.kernel(
        _combine_body,
        out_type=jax.ShapeDtypeStruct((TOKENS, D_MODEL), jnp.float32),
        mesh=mesh,
        scratch_types=(
            [pltpu.VMEM((_CCH, D_MODEL), jnp.float32)] * 4
            + [pltpu.VMEM((TPW, 16), jnp.float32)] * 2
            + [pltpu.VMEM((TPW,), jnp.int32)] * 2
            + [pltpu.SemaphoreType.DMA] * 5
        ),
    )(ys, pos_t, sb)


@jax.jit
def kernel(x, W1, b1, W2, b2, Wg, bg):
    B, N, D = x.shape
    x2 = x.reshape(B * N, D)
    pos01, sb, eot64 = _route(x2, Wg, bg)
    pos_t = pos01.T
    eot = eot64.reshape(64)
    xs = _dispatch(x2, pos_t)
    ys = _ffn(eot, xs, W1, b1.reshape(NUM_EXPERTS, 1, D_FF),
              W2, b2.reshape(NUM_EXPERTS, 1, D_MODEL))
    out2 = _combine(ys, pos_t, sb)
    return out2.reshape(B, N, D)


# speedup vs baseline: 1.2517x; 1.0067x over previous
import functools

import jax
import jax.numpy as jnp
from jax import lax
from jax.experimental import pallas as pl
from jax.experimental.pallas import tpu as pltpu
from jax.experimental.pallas import tpu_sc as plsc

D_MODEL = 1024
D_FF = 2048
NUM_EXPERTS = 8
TOP_K = 2
TOKENS = 4096
ROW_TILE = 256
NUM_TILES = 40
RSORT = NUM_TILES * ROW_TILE
NW = 32
TPW = TOKENS // NW


def _route_body(x_ref, wg_ref, bg_ref, pos_ref, sb_ref, eot_ref):
    xb = x_ref[...].astype(jnp.bfloat16)
    scores = jnp.dot(xb, wg_ref[...].astype(jnp.bfloat16),
                     preferred_element_type=jnp.float32) + bg_ref[...]
    cols = lax.broadcasted_iota(jnp.int32, (TOKENS, NUM_EXPERTS), 1)
    m = jnp.max(scores, axis=1, keepdims=True)
    p = jnp.exp(scores - m)
    p = p / jnp.sum(p, axis=1, keepdims=True)
    v0 = jnp.max(p, axis=1, keepdims=True)
    a0 = jnp.min(jnp.where(p >= v0, cols, NUM_EXPERTS), axis=1, keepdims=True)
    p1m = jnp.where(cols == a0, -jnp.inf, p)
    v1 = jnp.max(p1m, axis=1, keepdims=True)
    a1 = jnp.min(jnp.where(p1m >= v1, cols, NUM_EXPERTS), axis=1, keepdims=True)
    oh0 = (cols == a0).astype(jnp.float32)
    oh1 = (cols == a1).astype(jnp.float32)
    sel = oh0 + oh1
    c = sel
    d = 1
    while d < TOKENS:
        z = jnp.zeros((d, NUM_EXPERTS), jnp.float32)
        c = c + jnp.concatenate([z, c[:TOKENS - d, :]], axis=0)
        d *= 2
    rank = c - sel
    counts = jnp.sum(sel, axis=0, keepdims=True)
    cp = jnp.floor((counts + (ROW_TILE - 1)) / ROW_TILE) * ROW_TILE
    rr = lax.broadcasted_iota(jnp.int32, (NUM_EXPERTS, NUM_EXPERTS), 0)
    cc = lax.broadcasted_iota(jnp.int32, (NUM_EXPERTS, NUM_EXPERTS), 1)
    tri = (rr < cc).astype(jnp.float32)
    op = jnp.dot(cp, tri, preferred_element_type=jnp.float32)
    opend = op + cp
    ident = (rr == cc).astype(jnp.float32)
    opend_t = lax.dot_general(ident, opend, (((1,), (1,)), ((), ())),
                              preferred_element_type=jnp.float32)
    tile_start = lax.broadcasted_iota(
        jnp.int32, (NUM_EXPERTS, 64), 1).astype(jnp.float32) * ROW_TILE
    eot = jnp.sum((tile_start >= opend_t).astype(jnp.float32),
                  axis=0, keepdims=True)
    eot = jnp.minimum(eot, NUM_EXPERTS - 1.0)
    lanes64 = lax.broadcasted_iota(jnp.int32, (1, 64), 1)
    n_used = jnp.sum(cp) / ROW_TILE
    eot_ref[...] = jnp.where(lanes64 == 63, n_used, eot).astype(jnp.int32)
    posf = op + rank
    pw0 = jnp.sum(posf * oh0, axis=1, keepdims=True)
    pw1 = jnp.sum(posf * oh1, axis=1, keepdims=True)
    pos_ref[...] = jnp.concatenate([pw0, pw1], axis=1).astype(jnp.int32)
    s0 = jnp.sum(p * oh0, axis=1, keepdims=True)
    s1 = jnp.sum(p * oh1, axis=1, keepdims=True)
    sb_ref[...] = jnp.concatenate(
        [jnp.broadcast_to(s0, (TOKENS, 16)),
         jnp.broadcast_to(s1, (TOKENS, 16))], axis=0)


def _route(x2, Wg, bg):
    return pl.pallas_call(
        _route_body,
        out_shape=[
            jax.ShapeDtypeStruct((TOKENS, 2), jnp.int32),
            jax.ShapeDtypeStruct((2 * TOKENS, 16), jnp.float32),
            jax.ShapeDtypeStruct((1, 64), jnp.int32),
        ],
    )(x2, Wg, bg.reshape(1, NUM_EXPERTS))


_DCH = 32
_DNC = TPW // _DCH


def _dispatch_body(x_hbm, pos_hbm, xs_hbm, xb0, xb1, *rest):
    idx = rest[:2 * _DNC]
    isem, lsem0, lsem1, ssem0, ssem1 = rest[2 * _DNC:]
    xbufs = (xb0, xb1)
    lsems = (lsem0, lsem1)
    ssems = (ssem0, ssem1)
    wid = lax.axis_index("s") * 2 + lax.axis_index("c")
    base0 = wid * TPW
    icp = []
    for ci in range(_DNC):
        for k in range(2):
            icp.append(pltpu.async_copy(
                pos_hbm.at[k, pl.ds(base0 + ci * _DCH, _DCH)],
                idx[k * _DNC + ci], isem))
    loads = [None, None]
    scats = [None, None, None, None]
    for ci in range(2):
        loads[ci] = pltpu.async_copy(
            x_hbm.at[pl.ds(base0 + ci * _DCH, _DCH)], xbufs[ci], lsems[ci])
    for c in icp:
        c.wait()
    for ci in range(_DNC):
        b = ci % 2
        loads[b].wait()
        scats[2 * b] = pltpu.async_copy(
            xbufs[b], xs_hbm.at[idx[ci]], ssems[b])
        scats[2 * b + 1] = pltpu.async_copy(
            xbufs[b], xs_hbm.at[idx[_DNC + ci]], ssems[b])
        if ci + 2 < _DNC:
            scats[2 * b].wait()
            scats[2 * b + 1].wait()
            loads[b] = pltpu.async_copy(
                x_hbm.at[pl.ds(base0 + (ci + 2) * _DCH, _DCH)],
                xbufs[b], lsems[b])
    for b in range(2):
        scats[2 * b].wait()
        scats[2 * b + 1].wait()


def _dispatch(x2, pos_t):
    mesh = plsc.VectorSubcoreMesh(core_axis_name="c", subcore_axis_name="s")
    return pl.kernel(
        _dispatch_body,
        out_type=jax.ShapeDtypeStruct((RSORT, D_MODEL), jnp.float32),
        mesh=mesh,
        scratch_types=(
            [pltpu.VMEM((_DCH, D_MODEL), jnp.float32)] * 2
            + [pltpu.VMEM((_DCH,), jnp.int32)] * (2 * _DNC)
            + [pltpu.SemaphoreType.DMA] * 5
        ),
    )(x2, pos_t)


def _ffn_body(eot_ref, xs_ref, w1_ref, b1_ref, w2_ref, b2_ref, out_ref):
    i = pl.program_id(0)

    @pl.when(i < eot_ref[63])
    def _():
        h = jnp.dot(xs_ref[...], w1_ref[0], preferred_element_type=jnp.float32)
        h = jnp.maximum(h + b1_ref[0], 0.0)
        out_ref[...] = jnp.dot(h, w2_ref[0],
                               preferred_element_type=jnp.float32) + b2_ref[0]


def _ffn(eot, xs, W1b, b1r, W2b, b2r):
    grid_spec = pltpu.PrefetchScalarGridSpec(
        num_scalar_prefetch=1,
        grid=(NUM_TILES,),
        in_specs=[
            pl.BlockSpec((ROW_TILE, D_MODEL), lambda i, eot: (i, 0)),
            pl.BlockSpec((1, D_MODEL, D_FF), lambda i, eot: (eot[i], 0, 0)),
            pl.BlockSpec((1, 1, D_FF), lambda i, eot: (eot[i], 0, 0)),
            pl.BlockSpec((1, D_FF, D_MODEL), lambda i, eot: (eot[i], 0, 0)),
            pl.BlockSpec((1, 1, D_MODEL), lambda i, eot: (eot[i], 0, 0)),
        ],
        out_specs=pl.BlockSpec((ROW_TILE, D_MODEL), lambda i, eot: (i, 0)),
    )
    return pl.pallas_call(
        _ffn_body,
        grid_spec=grid_spec,
        out_shape=jax.ShapeDtypeStruct((RSORT, D_MODEL), jnp.float32),
        compiler_params=pltpu.CompilerParams(
            dimension_semantics=("arbitrary",),
        ),
    )(eot, xs, W1b, b1r, W2b, b2r)


_CCH = 16


_CNC = TPW // _CCH


def _combine_body(ys_hbm, pos_hbm, sb_hbm, out_hbm,
                  y0a, y1a, y0b, y1b,
                  sbuf0, sbuf1, idx0, idx1,
                  isem, gsa, gsb, osa, osb):
    wid = lax.axis_index("s") * 2 + lax.axis_index("c")
    base0 = wid * TPW
    i0 = pltpu.async_copy(pos_hbm.at[0, pl.ds(base0, TPW)], idx0, isem)
    i1 = pltpu.async_copy(pos_hbm.at[1, pl.ds(base0, TPW)], idx1, isem)
    i2 = pltpu.async_copy(sb_hbm.at[pl.ds(base0, TPW)], sbuf0, isem)
    i3 = pltpu.async_copy(sb_hbm.at[pl.ds(TOKENS + base0, TPW)], sbuf1, isem)
    i0.wait(); i1.wait(); i2.wait(); i3.wait()
    ybufs = ((y0a, y1a), (y0b, y1b))
    gsems = (gsa, gsb)
    osems = (osa, osb)
    gaths = [None, None]
    outs = [None, None]

    def gather(ci, b):
        sl = pl.ds(ci * _CCH, _CCH)
        g0 = pltpu.async_copy(ys_hbm.at[idx0.at[sl]], ybufs[b][0], gsems[b])
        g1 = pltpu.async_copy(ys_hbm.at[idx1.at[sl]], ybufs[b][1], gsems[b])
        return (g0, g1)

    gaths[0] = gather(0, 0)
    gaths[1] = gather(1, 1)
    for ci in range(_CNC):
        b = ci % 2
        gaths[b][0].wait()
        gaths[b][1].wait()
        y0, y1 = ybufs[b]

        def row_body(j, carry, y0=y0, y1=y1, ci=ci):
            sv0 = sbuf0[pl.ds(ci * _CCH + j, 1), pl.ds(0, 16)]
            sv1 = sbuf1[pl.ds(ci * _CCH + j, 1), pl.ds(0, 16)]
            s0v = sv0.reshape((16,))
            s1v = sv1.reshape((16,))
            for cc in range(D_MODEL // 16):
                sl2 = pl.ds(cc * 16, 16)
                y0[j, sl2] = y0[j, sl2] * s0v + y1[j, sl2] * s1v
            return carry

        lax.fori_loop(0, _CCH, row_body, 0)
        outs[b] = pltpu.async_copy(
            y0, out_hbm.at[pl.ds(base0 + ci * _CCH, _CCH)], osems[b])
        if ci + 2 < _CNC:
            outs[b].wait()
            gaths[b] = gather(ci + 2, b)
    for b in range(2):
        outs[b].wait()


def _combine(ys, pos_t, sb):
    mesh = plsc.VectorSubcoreMesh(core_axis_name="c", subcore_axis_name="s")
    return pl.kernel(
        _combine_body,
        out_type=jax.ShapeDtypeStruct((TOKENS, D_MODEL), jnp.float32),
        mesh=mesh,
        scratch_types=(
            [pltpu.VMEM((_CCH, D_MODEL), jnp.float32)] * 4
            + [pltpu.VMEM((TPW, 16), jnp.float32)] * 2
            + [pltpu.VMEM((TPW,), jnp.int32)] * 2
            + [pltpu.SemaphoreType.DMA] * 5
        ),
    )(ys, pos_t, sb)


@jax.jit
def kernel(x, W1, b1, W2, b2, Wg, bg):
    B, N, D = x.shape
    x2 = x.reshape(B * N, D)
    pos01, sb, eot64 = _route(x2, Wg, bg)
    pos_t = pos01.T
    eot = eot64.reshape(64)
    xs = _dispatch(x2, pos_t)
    ys = _ffn(eot, xs, W1, b1.reshape(NUM_EXPERTS, 1, D_FF),
              W2, b2.reshape(NUM_EXPERTS, 1, D_MODEL))
    out2 = _combine(ys, pos_t, sb)
    return out2.reshape(B, N, D)
